# baseline (speedup 1.0000x reference)
import functools

import jax
import jax.numpy as jnp
from jax import lax
from jax.experimental import pallas as pl
from jax.experimental.pallas import tpu as pltpu
from jax.experimental.pallas import tpu_sc as plsc

NC = 2
NS = 16
CHUNK = 128
IBLK = 16
LANES = 16


def _make_sc_agg(n_pad, d, e_pad, with_cnt):
    nchunks = e_pad // CHUNK
    assert nchunks % (NC * NS) == 0
    nl = nchunks // (NC * NS)
    assert nl % IBLK == 0
    nblk = nl // IBLK
    rows_per_tile = n_pad // NS

    out_type = [jax.ShapeDtypeStruct((NC, n_pad, d), jnp.float32)]
    if with_cnt:
        out_type.append(jax.ShapeDtypeStruct((NC, n_pad), jnp.float32))
    scratch = [
        pltpu.VMEM((IBLK, CHUNK), jnp.int32),
        pltpu.VMEM((IBLK, CHUNK), jnp.int32),
        pltpu.VMEM((IBLK, CHUNK), jnp.int32),
        pltpu.VMEM((IBLK, CHUNK), jnp.int32),
        pltpu.VMEM((CHUNK, d), jnp.float32),
        pltpu.VMEM((CHUNK, d), jnp.float32),
        pltpu.VMEM((CHUNK,), jnp.float32),
        pltpu.VMEM_SHARED((n_pad, d), jnp.float32),
        pltpu.VMEM_SHARED((n_pad,), jnp.float32),
        pltpu.SemaphoreType.DMA,
        pltpu.SemaphoreType.DMA,
        pltpu.SemaphoreType.DMA,
        pltpu.SemaphoreType.DMA,
        pltpu.SemaphoreType.DMA,
        pltpu.SemaphoreType.DMA,
    ]
    mesh = plsc.VectorSubcoreMesh(core_axis_name="c", subcore_axis_name="s")

    @functools.partial(pl.kernel, out_type=out_type, mesh=mesh,
                       scratch_types=scratch)
    def k(x_hbm, src_hbm, dst_hbm, zrows_hbm, zcnt_hbm, out_agg, *rest):
        if with_cnt:
            (out_cnt, sidx_a, didx_a, sidx_b, didx_b, rows_a, rows_b, ones,
             acc, cacc, sem_a, sem_b, sem_i, sem_z, sem_sa, sem_sb) = rest
        else:
            (sidx_a, didx_a, sidx_b, didx_b, rows_a, rows_b, ones,
             acc, cacc, sem_a, sem_b, sem_i, sem_z, sem_sa, sem_sb) = rest
        c = lax.axis_index("c")
        s = lax.axis_index("s")
        wid = c * NS + s
        chunk0 = wid * nl
        r0 = s * rows_per_tile
        ibufs = [(sidx_a, didx_a), (sidx_b, didx_b)]

        def load_block(b, p):
            sb, db = ibufs[p]
            pltpu.async_copy(src_hbm.at[pl.ds(chunk0 + b * IBLK, IBLK)],
                             sb, sem_i)
            pltpu.async_copy(dst_hbm.at[pl.ds(chunk0 + b * IBLK, IBLK)],
                             db, sem_i)

        def wait_block(b, p):
            sb, db = ibufs[p]
            pltpu.make_async_copy(
                src_hbm.at[pl.ds(chunk0 + b * IBLK, IBLK)], sb, sem_i).wait()
            pltpu.make_async_copy(
                dst_hbm.at[pl.ds(chunk0 + b * IBLK, IBLK)], db, sem_i).wait()

        zdesc = pltpu.async_copy(zrows_hbm.at[pl.ds(r0, rows_per_tile)],
                                 acc.at[pl.ds(r0, rows_per_tile)], sem_z)
        if with_cnt:
            zcdesc = pltpu.async_copy(zcnt_hbm.at[pl.ds(r0, rows_per_tile)],
                                      cacc.at[pl.ds(r0, rows_per_tile)], sem_z)
        load_block(0, 0)
        if nblk > 1:
            load_block(1, 1)
        if with_cnt:
            for i in range(CHUNK // LANES):
                ones[pl.ds(i * LANES, LANES)] = jnp.ones((LANES,), jnp.float32)
        zdesc.wait()
        if with_cnt:
            zcdesc.wait()
        plsc.subcore_barrier()

        for b in range(nblk):
            p = b % 2
            sidx, didx = ibufs[p]
            wait_block(b, p)

            def gather(j, buf, sem):
                pltpu.async_copy(x_hbm.at[sidx.at[j]], buf, sem)

            gather(0, rows_a, sem_a)
            gather(1, rows_b, sem_b)

            def body(i, carry2, sidx=sidx, didx=didx, gather=gather):
                ja = 2 * i
                jb = 2 * i + 1
                pltpu.make_async_copy(
                    x_hbm.at[sidx.at[ja]], rows_a, sem_a).wait()
                dsc_a = pltpu.async_copy(
                    rows_a, acc.at[didx.at[ja]], sem_sa, add=True)
                if with_cnt:
                    dsc_ca = pltpu.async_copy(
                        ones, cacc.at[didx.at[ja]], sem_sa, add=True)
                pltpu.make_async_copy(
                    x_hbm.at[sidx.at[jb]], rows_b, sem_b).wait()
                dsc_b = pltpu.async_copy(
                    rows_b, acc.at[didx.at[jb]], sem_sb, add=True)
                if with_cnt:
                    dsc_cb = pltpu.async_copy(
                        ones, cacc.at[didx.at[jb]], sem_sb, add=True)
                dsc_a.wait()
                if with_cnt:
                    dsc_ca.wait()

                @pl.when(ja + 2 < IBLK)
                def _():
                    gather(ja + 2, rows_a, sem_a)

                dsc_b.wait()
                if with_cnt:
                    dsc_cb.wait()

                @pl.when(jb + 2 < IBLK)
                def _():
                    gather(jb + 2, rows_b, sem_b)

                return carry2

            lax.fori_loop(0, IBLK // 2, body, 0)
            if b + 2 < nblk:
                load_block(b + 2, p)

        plsc.subcore_barrier()
        pltpu.sync_copy(acc.at[pl.ds(r0, rows_per_tile)],
                        out_agg.at[c, pl.ds(r0, rows_per_tile)])
        if with_cnt:
            pltpu.sync_copy(cacc.at[pl.ds(r0, rows_per_tile)],
                            out_cnt.at[c, pl.ds(r0, rows_per_tile)])

    return k


def _tc_layer(final, agg0, agg1, cnt0, cnt1, xin, wl_t, wr_t, b, out_rows):
    n_pad, d = agg0.shape
    blk = 1024

    def body(a0, a1, c0, c1, x, wl, wr, bb, out):
        cnt = jnp.maximum(c0[...] + c1[...], 1.0)
        mean = (a0[...] + a1[...]) / cnt
        z = (jnp.dot(mean, wl[...], preferred_element_type=jnp.float32)
             + bb[...]
             + jnp.dot(x[...], wr[...], preferred_element_type=jnp.float32))
        if final:
            m = jnp.max(z, axis=-1, keepdims=True)
            ez = jnp.exp(z - m)
            out[...] = (z - m) - jnp.log(jnp.sum(ez, axis=-1, keepdims=True))
        else:
            out[...] = jnp.maximum(z, 0.0)

    row_spec = pl.BlockSpec((blk, d), lambda i: (i, 0))
    col_spec = pl.BlockSpec((blk, 1), lambda i: (i, 0))
    full_spec = pl.BlockSpec((d, d), lambda i: (0, 0))
    b_spec = pl.BlockSpec((1, d), lambda i: (0, 0))
    return pl.pallas_call(
        body,
        grid=(n_pad // blk,),
        in_specs=[row_spec, row_spec, col_spec, col_spec, row_spec,
                  full_spec, full_spec, b_spec],
        out_specs=row_spec,
        out_shape=jax.ShapeDtypeStruct((out_rows, d), jnp.float32),
    )(agg0, agg1, cnt0, cnt1, xin, wl_t, wr_t, b)


def kernel(x, edge_index, W1l, b1, W1r, W2l, b2, W2r):
    n, d = x.shape
    e = edge_index.shape[1]
    n_pad = -(-(n + 1) // 2048) * 2048
    span = NC * NS * CHUNK
    nl = -(-e // span)
    nl = -(-nl // IBLK) * IBLK
    e_pad = nl * span

    src = edge_index[0].astype(jnp.int32)
    dst = edge_index[1].astype(jnp.int32)
    if e_pad != e:
        pad = e_pad - e
        r = jnp.arange(pad, dtype=jnp.int32)
        src = jnp.concatenate([src, r % n])
        dst = jnp.concatenate([dst, n + r % (n_pad - n)])
    src = src.reshape(e_pad // CHUNK, CHUNK)
    dst = dst.reshape(e_pad // CHUNK, CHUNK)

    zrows = jnp.zeros((n_pad, d), jnp.float32)
    zcnt = jnp.zeros((n_pad,), jnp.float32)

    sc_agg_cnt = _make_sc_agg(n_pad, d, e_pad, True)
    sc_agg = _make_sc_agg(n_pad, d, e_pad, False)

    aggp, cntp = sc_agg_cnt(x, src, dst, zrows, zcnt)
    cnt0 = cntp[0].reshape(n_pad, 1)
    cnt1 = cntp[1].reshape(n_pad, 1)
    h = _tc_layer(False, aggp[0], aggp[1], cnt0, cnt1, x,
                  W1l.T, W1r.T, b1.reshape(1, d), n_pad)
    (aggp2,) = sc_agg(h, src, dst, zrows, zcnt)
    return _tc_layer(True, aggp2[0], aggp2[1], cnt0, cnt1, h,
                     W2l.T, W2r.T, b2.reshape(1, d), n)


# speedup vs baseline: 1.2084x; 1.2084x over previous
import functools

import jax
import jax.numpy as jnp
from jax import lax
from jax.experimental import pallas as pl
from jax.experimental.pallas import tpu as pltpu
from jax.experimental.pallas import tpu_sc as plsc

NC = 2
NS = 16
CHUNK = 128
IBLK = 16
LANES = 16


def _make_sc_agg(n_pad, d, e_pad, with_cnt):
    nchunks = e_pad // CHUNK
    assert nchunks % (NC * NS) == 0
    nl = nchunks // (NC * NS)
    assert nl % IBLK == 0
    nblk = nl // IBLK
    rows_per_tile = n_pad // NS

    out_type = [jax.ShapeDtypeStruct((NC, n_pad, d), jnp.float32)]
    if with_cnt:
        out_type.append(jax.ShapeDtypeStruct((NC, n_pad), jnp.float32))
    scratch = [
        pltpu.VMEM((IBLK, CHUNK), jnp.int32),
        pltpu.VMEM((IBLK, CHUNK), jnp.int32),
        pltpu.VMEM((IBLK, CHUNK), jnp.int32),
        pltpu.VMEM((IBLK, CHUNK), jnp.int32),
        pltpu.VMEM((CHUNK, d), jnp.float32),
        pltpu.VMEM((CHUNK, d), jnp.float32),
        pltpu.VMEM((CHUNK,), jnp.float32),
        pltpu.VMEM_SHARED((n_pad, d), jnp.float32),
        pltpu.VMEM_SHARED((n_pad,), jnp.float32),
        pltpu.SemaphoreType.DMA,
        pltpu.SemaphoreType.DMA,
        pltpu.SemaphoreType.DMA,
        pltpu.SemaphoreType.DMA,
        pltpu.SemaphoreType.DMA,
        pltpu.SemaphoreType.DMA,
    ]
    mesh = plsc.VectorSubcoreMesh(core_axis_name="c", subcore_axis_name="s")

    @functools.partial(pl.kernel, out_type=out_type, mesh=mesh,
                       scratch_types=scratch)
    def k(x_hbm, src_hbm, dst_hbm, zrows_hbm, zcnt_hbm, out_agg, *rest):
        if with_cnt:
            (out_cnt, sidx_a, didx_a, sidx_b, didx_b, rows_a, rows_b, ones,
             acc, cacc, sem_a, sem_b, sem_i, sem_z, sem_sa, sem_sb) = rest
        else:
            (sidx_a, didx_a, sidx_b, didx_b, rows_a, rows_b, ones,
             acc, cacc, sem_a, sem_b, sem_i, sem_z, sem_sa, sem_sb) = rest
        c = lax.axis_index("c")
        s = lax.axis_index("s")
        wid = c * NS + s
        chunk0 = wid * nl
        r0 = s * rows_per_tile
        ibufs = [(sidx_a, didx_a), (sidx_b, didx_b)]

        def load_block(b, p):
            sb, db = ibufs[p]
            pltpu.async_copy(src_hbm.at[pl.ds(chunk0 + b * IBLK, IBLK)],
                             sb, sem_i)
            pltpu.async_copy(dst_hbm.at[pl.ds(chunk0 + b * IBLK, IBLK)],
                             db, sem_i)

        def wait_block(b, p):
            sb, db = ibufs[p]
            pltpu.make_async_copy(
                src_hbm.at[pl.ds(chunk0 + b * IBLK, IBLK)], sb, sem_i).wait()
            pltpu.make_async_copy(
                dst_hbm.at[pl.ds(chunk0 + b * IBLK, IBLK)], db, sem_i).wait()

        zdesc = pltpu.async_copy(zrows_hbm.at[pl.ds(r0, rows_per_tile)],
                                 acc.at[pl.ds(r0, rows_per_tile)], sem_z)
        if with_cnt:
            zcdesc = pltpu.async_copy(zcnt_hbm.at[pl.ds(r0, rows_per_tile)],
                                      cacc.at[pl.ds(r0, rows_per_tile)], sem_z)
        load_block(0, 0)
        if nblk > 1:
            load_block(1, 1)
        if with_cnt:
            for i in range(CHUNK // LANES):
                ones[pl.ds(i * LANES, LANES)] = jnp.ones((LANES,), jnp.float32)
        zdesc.wait()
        if with_cnt:
            zcdesc.wait()
        plsc.subcore_barrier()

        for b in range(nblk):
            p = b % 2
            sidx, didx = ibufs[p]
            wait_block(b, p)

            def gather(j, buf, sem):
                pltpu.async_copy(x_hbm.at[sidx.at[j]], buf, sem)

            gather(0, rows_a, sem_a)

            def body(i, carry2, sidx=sidx, didx=didx, gather=gather):
                ja = 2 * i
                jb = 2 * i + 1
                gather(jb, rows_b, sem_b)
                pltpu.make_async_copy(
                    x_hbm.at[sidx.at[ja]], rows_a, sem_a).wait()
                pltpu.sync_copy(rows_a, acc.at[didx.at[ja]], add=True)
                if with_cnt:
                    pltpu.sync_copy(ones, cacc.at[didx.at[ja]], add=True)

                @pl.when(ja + 2 < IBLK)
                def _():
                    gather(ja + 2, rows_a, sem_a)

                pltpu.make_async_copy(
                    x_hbm.at[sidx.at[jb]], rows_b, sem_b).wait()
                pltpu.sync_copy(rows_b, acc.at[didx.at[jb]], add=True)
                if with_cnt:
                    pltpu.sync_copy(ones, cacc.at[didx.at[jb]], add=True)
                return carry2

            lax.fori_loop(0, IBLK // 2, body, 0)
            if b + 2 < nblk:
                load_block(b + 2, p)

        plsc.subcore_barrier()
        pltpu.sync_copy(acc.at[pl.ds(r0, rows_per_tile)],
                        out_agg.at[c, pl.ds(r0, rows_per_tile)])
        if with_cnt:
            pltpu.sync_copy(cacc.at[pl.ds(r0, rows_per_tile)],
                            out_cnt.at[c, pl.ds(r0, rows_per_tile)])

    return k


def _tc_layer(final, agg0, agg1, cnt0, cnt1, xin, wl_t, wr_t, b, out_rows):
    n_pad, d = agg0.shape
    blk = 1024

    def body(a0, a1, c0, c1, x, wl, wr, bb, out):
        cnt = jnp.maximum(c0[...] + c1[...], 1.0)
        mean = (a0[...] + a1[...]) / cnt
        z = (jnp.dot(mean, wl[...], preferred_element_type=jnp.float32)
             + bb[...]
             + jnp.dot(x[...], wr[...], preferred_element_type=jnp.float32))
        if final:
            m = jnp.max(z, axis=-1, keepdims=True)
            ez = jnp.exp(z - m)
            out[...] = (z - m) - jnp.log(jnp.sum(ez, axis=-1, keepdims=True))
        else:
            out[...] = jnp.maximum(z, 0.0)

    row_spec = pl.BlockSpec((blk, d), lambda i: (i, 0))
    col_spec = pl.BlockSpec((blk, 1), lambda i: (i, 0))
    full_spec = pl.BlockSpec((d, d), lambda i: (0, 0))
    b_spec = pl.BlockSpec((1, d), lambda i: (0, 0))
    return pl.pallas_call(
        body,
        grid=(n_pad // blk,),
        in_specs=[row_spec, row_spec, col_spec, col_spec, row_spec,
                  full_spec, full_spec, b_spec],
        out_specs=row_spec,
        out_shape=jax.ShapeDtypeStruct((out_rows, d), jnp.float32),
    )(agg0, agg1, cnt0, cnt1, xin, wl_t, wr_t, b)


def kernel(x, edge_index, W1l, b1, W1r, W2l, b2, W2r):
    n, d = x.shape
    e = edge_index.shape[1]
    n_pad = -(-(n + 1) // 2048) * 2048
    span = NC * NS * CHUNK
    nl = -(-e // span)
    nl = -(-nl // IBLK) * IBLK
    e_pad = nl * span

    src = edge_index[0].astype(jnp.int32)
    dst = edge_index[1].astype(jnp.int32)
    if e_pad != e:
        pad = e_pad - e
        r = jnp.arange(pad, dtype=jnp.int32)
        src = jnp.concatenate([src, r % n])
        dst = jnp.concatenate([dst, n + r % (n_pad - n)])
    src = src.reshape(e_pad // CHUNK, CHUNK)
    dst = dst.reshape(e_pad // CHUNK, CHUNK)

    zrows = jnp.zeros((n_pad, d), jnp.float32)
    zcnt = jnp.zeros((n_pad,), jnp.float32)

    sc_agg_cnt = _make_sc_agg(n_pad, d, e_pad, True)
    sc_agg = _make_sc_agg(n_pad, d, e_pad, False)

    aggp, cntp = sc_agg_cnt(x, src, dst, zrows, zcnt)
    cnt0 = cntp[0].reshape(n_pad, 1)
    cnt1 = cntp[1].reshape(n_pad, 1)
    h = _tc_layer(False, aggp[0], aggp[1], cnt0, cnt1, x,
                  W1l.T, W1r.T, b1.reshape(1, d), n_pad)
    (aggp2,) = sc_agg(h, src, dst, zrows, zcnt)
    return _tc_layer(True, aggp2[0], aggp2[1], cnt0, cnt1, h,
                     W2l.T, W2r.T, b2.reshape(1, d), n)


# speedup vs baseline: 1.2760x; 1.0560x over previous
import functools

import jax
import jax.numpy as jnp
from jax import lax
from jax.experimental import pallas as pl
from jax.experimental.pallas import tpu as pltpu
from jax.experimental.pallas import tpu_sc as plsc

NC = 2
NS = 16
CHUNK = 128
IBLK = 16
LANES = 16


def _make_sc_agg(n_pad, d, e_pad, with_cnt):
    nchunks = e_pad // CHUNK
    assert nchunks % (NC * NS) == 0
    nl = nchunks // (NC * NS)
    assert nl % IBLK == 0
    nblk = nl // IBLK
    rows_per_tile = n_pad // NS

    out_type = [jax.ShapeDtypeStruct((NC, n_pad, d), jnp.float32)]
    if with_cnt:
        out_type.append(jax.ShapeDtypeStruct((NC, n_pad), jnp.float32))
    scratch = [
        pltpu.VMEM((IBLK, CHUNK), jnp.int32),
        pltpu.VMEM((IBLK, CHUNK), jnp.int32),
        pltpu.VMEM((IBLK, CHUNK), jnp.int32),
        pltpu.VMEM((IBLK, CHUNK), jnp.int32),
        pltpu.VMEM((CHUNK, d), jnp.float32),
        pltpu.VMEM((CHUNK, d), jnp.float32),
        pltpu.VMEM((CHUNK,), jnp.float32),
        pltpu.VMEM_SHARED((n_pad, d), jnp.float32),
        pltpu.VMEM_SHARED((n_pad,), jnp.float32),
        pltpu.SemaphoreType.DMA,
        pltpu.SemaphoreType.DMA,
        pltpu.SemaphoreType.DMA,
        pltpu.SemaphoreType.DMA,
        pltpu.SemaphoreType.DMA,
        pltpu.SemaphoreType.DMA,
    ]
    mesh = plsc.VectorSubcoreMesh(core_axis_name="c", subcore_axis_name="s")

    @functools.partial(pl.kernel, out_type=out_type, mesh=mesh,
                       scratch_types=scratch)
    def k(x_hbm, src_hbm, dst_hbm, zrows_hbm, zcnt_hbm, out_agg, *rest):
        if with_cnt:
            (out_cnt, sidx_a, didx_a, sidx_b, didx_b, rows_a, rows_b, ones,
             acc, cacc, sem_a, sem_b, sem_i, sem_z, sem_sa, sem_sb) = rest
        else:
            (sidx_a, didx_a, sidx_b, didx_b, rows_a, rows_b, ones,
             acc, cacc, sem_a, sem_b, sem_i, sem_z, sem_sa, sem_sb) = rest
        c = lax.axis_index("c")
        s = lax.axis_index("s")
        wid = c * NS + s
        chunk0 = wid * nl
        r0 = s * rows_per_tile
        ibufs = [(sidx_a, didx_a), (sidx_b, didx_b)]

        def load_block(b, p):
            sb, db = ibufs[p]
            pltpu.async_copy(src_hbm.at[pl.ds(chunk0 + b * IBLK, IBLK)],
                             sb, sem_i)
            pltpu.async_copy(dst_hbm.at[pl.ds(chunk0 + b * IBLK, IBLK)],
                             db, sem_i)

        def wait_block(b, p):
            sb, db = ibufs[p]
            pltpu.make_async_copy(
                src_hbm.at[pl.ds(chunk0 + b * IBLK, IBLK)], sb, sem_i).wait()
            pltpu.make_async_copy(
                dst_hbm.at[pl.ds(chunk0 + b * IBLK, IBLK)], db, sem_i).wait()

        zdesc = pltpu.async_copy(zrows_hbm.at[pl.ds(r0, rows_per_tile)],
                                 acc.at[pl.ds(r0, rows_per_tile)], sem_z)
        if with_cnt:
            zcdesc = pltpu.async_copy(zcnt_hbm.at[pl.ds(r0, rows_per_tile)],
                                      cacc.at[pl.ds(r0, rows_per_tile)], sem_z)
        load_block(0, 0)
        if nblk > 1:
            load_block(1, 1)
        if with_cnt:
            for i in range(CHUNK // LANES):
                ones[pl.ds(i * LANES, LANES)] = jnp.ones((LANES,), jnp.float32)
        zdesc.wait()
        if with_cnt:
            zcdesc.wait()
        plsc.subcore_barrier()

        for b in range(nblk):
            p = b % 2
            sidx, didx = ibufs[p]
            wait_block(b, p)

            def gather(j, buf, sem):
                pltpu.async_copy(x_hbm.at[sidx.at[j]], buf, sem)

            gather(0, rows_a, sem_a)

            def body(i, carry2, sidx=sidx, didx=didx, gather=gather):
                ja = 2 * i
                jb = 2 * i + 1
                gather(jb, rows_b, sem_b)
                pltpu.make_async_copy(
                    x_hbm.at[sidx.at[ja]], rows_a, sem_a).wait()
                pltpu.sync_copy(rows_a, acc.at[didx.at[ja]], add=True)
                if with_cnt:
                    pltpu.sync_copy(ones, cacc.at[didx.at[ja]], add=True)

                @pl.when(ja + 2 < IBLK)
                def _():
                    gather(ja + 2, rows_a, sem_a)

                pltpu.make_async_copy(
                    x_hbm.at[sidx.at[jb]], rows_b, sem_b).wait()
                pltpu.sync_copy(rows_b, acc.at[didx.at[jb]], add=True)
                if with_cnt:
                    pltpu.sync_copy(ones, cacc.at[didx.at[jb]], add=True)
                return carry2

            lax.fori_loop(0, IBLK // 2, body, 0)
            if b + 2 < nblk:
                load_block(b + 2, p)

        plsc.subcore_barrier()
        pltpu.sync_copy(acc.at[pl.ds(r0, rows_per_tile)],
                        out_agg.at[c, pl.ds(r0, rows_per_tile)])
        if with_cnt:
            pltpu.sync_copy(cacc.at[pl.ds(r0, rows_per_tile)],
                            out_cnt.at[c, pl.ds(r0, rows_per_tile)])

    return k


def _tc_layer(final, aggp, cnt0, cnt1, xin, wl_t, wr_t, b, out_rows):
    _, n_pad, d = aggp.shape
    blk = 2048

    def body(a0, a1, c0, c1, x, wl, wr, bb, out):
        cnt = jnp.maximum(c0[...] + c1[...], 1.0)
        mean = (a0[0] + a1[0]) / cnt
        z = (jnp.dot(mean, wl[...], preferred_element_type=jnp.float32)
             + bb[...]
             + jnp.dot(x[...], wr[...], preferred_element_type=jnp.float32))
        if final:
            m = jnp.max(z, axis=-1, keepdims=True)
            ez = jnp.exp(z - m)
            out[...] = (z - m) - jnp.log(jnp.sum(ez, axis=-1, keepdims=True))
        else:
            out[...] = jnp.maximum(z, 0.0)

    agg0_spec = pl.BlockSpec((1, blk, d), lambda i: (0, i, 0))
    agg1_spec = pl.BlockSpec((1, blk, d), lambda i: (1, i, 0))
    row_spec = pl.BlockSpec((blk, d), lambda i: (i, 0))
    col_spec = pl.BlockSpec((blk, 1), lambda i: (i, 0))
    full_spec = pl.BlockSpec((d, d), lambda i: (0, 0))
    b_spec = pl.BlockSpec((1, d), lambda i: (0, 0))
    return pl.pallas_call(
        body,
        grid=(n_pad // blk,),
        in_specs=[agg0_spec, agg1_spec, col_spec, col_spec, row_spec,
                  full_spec, full_spec, b_spec],
        out_specs=row_spec,
        out_shape=jax.ShapeDtypeStruct((out_rows, d), jnp.float32),
    )(aggp, aggp, cnt0, cnt1, xin, wl_t, wr_t, b)


def kernel(x, edge_index, W1l, b1, W1r, W2l, b2, W2r):
    n, d = x.shape
    e = edge_index.shape[1]
    n_pad = -(-(n + 1) // 2048) * 2048
    span = NC * NS * CHUNK
    nl = -(-e // span)
    nl = -(-nl // IBLK) * IBLK
    e_pad = nl * span

    src = edge_index[0].astype(jnp.int32)
    dst = edge_index[1].astype(jnp.int32)
    if e_pad != e:
        pad = e_pad - e
        r = jnp.arange(pad, dtype=jnp.int32)
        src = jnp.concatenate([src, r % n])
        dst = jnp.concatenate([dst, n + r % (n_pad - n)])
    src = src.reshape(e_pad // CHUNK, CHUNK)
    dst = dst.reshape(e_pad // CHUNK, CHUNK)

    zrows = jnp.zeros((n_pad, d), jnp.float32)
    zcnt = jnp.zeros((n_pad,), jnp.float32)

    sc_agg_cnt = _make_sc_agg(n_pad, d, e_pad, True)
    sc_agg = _make_sc_agg(n_pad, d, e_pad, False)

    aggp, cntp = sc_agg_cnt(x, src, dst, zrows, zcnt)
    cnt0 = cntp[0].reshape(n_pad, 1)
    cnt1 = cntp[1].reshape(n_pad, 1)
    h = _tc_layer(False, aggp, cnt0, cnt1, x,
                  W1l.T, W1r.T, b1.reshape(1, d), n_pad)
    (aggp2,) = sc_agg(h, src, dst, zrows, zcnt)
    return _tc_layer(True, aggp2, cnt0, cnt1, h,
                     W2l.T, W2r.T, b2.reshape(1, d), n)


# speedup vs baseline: 1.3717x; 1.0749x over previous
import functools

import jax
import jax.numpy as jnp
from jax import lax
from jax.experimental import pallas as pl
from jax.experimental.pallas import tpu as pltpu
from jax.experimental.pallas import tpu_sc as plsc

NC = 2
NS = 16
CHUNK = 128
IBLK = 16
LANES = 16


def _make_sc_agg(n_pad, d, e_pad, with_cnt):
    nchunks = e_pad // CHUNK
    assert nchunks % (NC * NS) == 0
    nl = nchunks // (NC * NS)
    assert nl % IBLK == 0
    nblk = nl // IBLK
    rows_per_tile = n_pad // NS

    dt = jnp.bfloat16
    out_type = [jax.ShapeDtypeStruct((NC, n_pad, d), dt)]
    if with_cnt:
        out_type.append(jax.ShapeDtypeStruct((NC, n_pad), jnp.float32))
    scratch = [
        pltpu.VMEM((IBLK, CHUNK), jnp.int32),
        pltpu.VMEM((IBLK, CHUNK), jnp.int32),
        pltpu.VMEM((IBLK, CHUNK), jnp.int32),
        pltpu.VMEM((IBLK, CHUNK), jnp.int32),
        pltpu.VMEM((CHUNK, d), dt),
        pltpu.VMEM((CHUNK, d), dt),
        pltpu.VMEM((CHUNK,), jnp.float32),
        pltpu.VMEM_SHARED((n_pad, d), dt),
        pltpu.VMEM_SHARED((n_pad,), jnp.float32),
        pltpu.SemaphoreType.DMA,
        pltpu.SemaphoreType.DMA,
        pltpu.SemaphoreType.DMA,
        pltpu.SemaphoreType.DMA,
        pltpu.SemaphoreType.DMA,
        pltpu.SemaphoreType.DMA,
    ]
    mesh = plsc.VectorSubcoreMesh(core_axis_name="c", subcore_axis_name="s")

    @functools.partial(
        pl.kernel, out_type=out_type, mesh=mesh, scratch_types=scratch,
        compiler_params=pltpu.CompilerParams(use_tc_tiling_on_sc=False))
    def k(x_hbm, src_hbm, dst_hbm, zrows_hbm, zcnt_hbm, out_agg, *rest):
        if with_cnt:
            (out_cnt, sidx_a, didx_a, sidx_b, didx_b, rows_a, rows_b, ones,
             acc, cacc, sem_a, sem_b, sem_i, sem_z, sem_sa, sem_sb) = rest
        else:
            (sidx_a, didx_a, sidx_b, didx_b, rows_a, rows_b, ones,
             acc, cacc, sem_a, sem_b, sem_i, sem_z, sem_sa, sem_sb) = rest
        c = lax.axis_index("c")
        s = lax.axis_index("s")
        wid = c * NS + s
        chunk0 = wid * nl
        r0 = s * rows_per_tile
        ibufs = [(sidx_a, didx_a), (sidx_b, didx_b)]

        def load_block(b, p):
            sb, db = ibufs[p]
            pltpu.async_copy(src_hbm.at[pl.ds(chunk0 + b * IBLK, IBLK)],
                             sb, sem_i)
            pltpu.async_copy(dst_hbm.at[pl.ds(chunk0 + b * IBLK, IBLK)],
                             db, sem_i)

        def wait_block(b, p):
            sb, db = ibufs[p]
            pltpu.make_async_copy(
                src_hbm.at[pl.ds(chunk0 + b * IBLK, IBLK)], sb, sem_i).wait()
            pltpu.make_async_copy(
                dst_hbm.at[pl.ds(chunk0 + b * IBLK, IBLK)], db, sem_i).wait()

        zdesc = pltpu.async_copy(zrows_hbm.at[pl.ds(r0, rows_per_tile)],
                                 acc.at[pl.ds(r0, rows_per_tile)], sem_z)
        if with_cnt:
            zcdesc = pltpu.async_copy(zcnt_hbm.at[pl.ds(r0, rows_per_tile)],
                                      cacc.at[pl.ds(r0, rows_per_tile)], sem_z)
        load_block(0, 0)
        if nblk > 1:
            load_block(1, 1)
        if with_cnt:
            for i in range(CHUNK // LANES):
                ones[pl.ds(i * LANES, LANES)] = jnp.ones((LANES,), jnp.float32)
        zdesc.wait()
        if with_cnt:
            zcdesc.wait()
        plsc.subcore_barrier()

        for b in range(nblk):
            p = b % 2
            sidx, didx = ibufs[p]
            wait_block(b, p)

            def gather(j, buf, sem):
                pltpu.async_copy(x_hbm.at[sidx.at[j]], buf, sem)

            gather(0, rows_a, sem_a)

            def body(i, carry2, sidx=sidx, didx=didx, gather=gather):
                ja = 2 * i
                jb = 2 * i + 1
                gather(jb, rows_b, sem_b)
                pltpu.make_async_copy(
                    x_hbm.at[sidx.at[ja]], rows_a, sem_a).wait()
                pltpu.sync_copy(rows_a, acc.at[didx.at[ja]], add=True)
                if with_cnt:
                    pltpu.sync_copy(ones, cacc.at[didx.at[ja]], add=True)

                @pl.when(ja + 2 < IBLK)
                def _():
                    gather(ja + 2, rows_a, sem_a)

                pltpu.make_async_copy(
                    x_hbm.at[sidx.at[jb]], rows_b, sem_b).wait()
                pltpu.sync_copy(rows_b, acc.at[didx.at[jb]], add=True)
                if with_cnt:
                    pltpu.sync_copy(ones, cacc.at[didx.at[jb]], add=True)
                return carry2

            lax.fori_loop(0, IBLK // 2, body, 0)
            if b + 2 < nblk:
                load_block(b + 2, p)

        plsc.subcore_barrier()
        pltpu.sync_copy(acc.at[pl.ds(r0, rows_per_tile)],
                        out_agg.at[c, pl.ds(r0, rows_per_tile)])
        if with_cnt:
            pltpu.sync_copy(cacc.at[pl.ds(r0, rows_per_tile)],
                            out_cnt.at[c, pl.ds(r0, rows_per_tile)])

    return k


def _tc_layer(final, aggp, cnt0, cnt1, xin, wl_t, wr_t, b, out_rows):
    _, n_pad, d = aggp.shape
    blk = 2048
    out_dtype = jnp.float32 if final else jnp.bfloat16

    def body(a0, a1, c0, c1, x, wl, wr, bb, out):
        cnt = jnp.maximum(c0[...] + c1[...], 1.0)
        mean = (a0[0].astype(jnp.float32) + a1[0].astype(jnp.float32)) / cnt
        z = (jnp.dot(mean, wl[...], preferred_element_type=jnp.float32)
             + bb[...]
             + jnp.dot(x[...], wr[...], preferred_element_type=jnp.float32))
        if final:
            m = jnp.max(z, axis=-1, keepdims=True)
            ez = jnp.exp(z - m)
            out[...] = (z - m) - jnp.log(jnp.sum(ez, axis=-1, keepdims=True))
        else:
            out[...] = jnp.maximum(z, 0.0).astype(out_dtype)

    agg0_spec = pl.BlockSpec((1, blk, d), lambda i: (0, i, 0))
    agg1_spec = pl.BlockSpec((1, blk, d), lambda i: (1, i, 0))
    row_spec = pl.BlockSpec((blk, d), lambda i: (i, 0))
    col_spec = pl.BlockSpec((blk, 1), lambda i: (i, 0))
    full_spec = pl.BlockSpec((d, d), lambda i: (0, 0))
    b_spec = pl.BlockSpec((1, d), lambda i: (0, 0))
    return pl.pallas_call(
        body,
        grid=(n_pad // blk,),
        in_specs=[agg0_spec, agg1_spec, col_spec, col_spec, row_spec,
                  full_spec, full_spec, b_spec],
        out_specs=row_spec,
        out_shape=jax.ShapeDtypeStruct((out_rows, d), out_dtype),
    )(aggp, aggp, cnt0, cnt1, xin, wl_t, wr_t, b)


def kernel(x, edge_index, W1l, b1, W1r, W2l, b2, W2r):
    n, d = x.shape
    e = edge_index.shape[1]
    n_pad = -(-(n + 1) // 2048) * 2048
    span = NC * NS * CHUNK
    nl = -(-e // span)
    nl = -(-nl // IBLK) * IBLK
    e_pad = nl * span

    src = edge_index[0].astype(jnp.int32)
    dst = edge_index[1].astype(jnp.int32)
    if e_pad != e:
        pad = e_pad - e
        r = jnp.arange(pad, dtype=jnp.int32)
        src = jnp.concatenate([src, r % n])
        dst = jnp.concatenate([dst, n + r % (n_pad - n)])
    src = src.reshape(e_pad // CHUNK, CHUNK)
    dst = dst.reshape(e_pad // CHUNK, CHUNK)

    zrows = jnp.zeros((n_pad, d), jnp.bfloat16)
    zcnt = jnp.zeros((n_pad,), jnp.float32)
    x_bf = x.astype(jnp.bfloat16)

    sc_agg_cnt = _make_sc_agg(n_pad, d, e_pad, True)
    sc_agg = _make_sc_agg(n_pad, d, e_pad, False)

    aggp, cntp = sc_agg_cnt(x_bf, src, dst, zrows, zcnt)
    cnt0 = cntp[0].reshape(n_pad, 1)
    cnt1 = cntp[1].reshape(n_pad, 1)
    h_bf = _tc_layer(False, aggp, cnt0, cnt1, x,
                     W1l.T, W1r.T, b1.reshape(1, d), n_pad)
    (aggp2,) = sc_agg(h_bf, src, dst, zrows, zcnt)
    return _tc_layer(True, aggp2, cnt0, cnt1, h_bf,
                     W2l.T, W2r.T, b2.reshape(1, d), n)
